# baseline (speedup 1.0000x reference)
import functools

import jax
import jax.numpy as jnp
from jax import lax
from jax.experimental import pallas as pl
from jax.experimental.pallas import tpu as pltpu
from jax.experimental.pallas import tpu_sc as plsc

_B, _S, _D = 4, 2048, 1024
_NC, _NS = 2, 16
_NW = _NC * _NS
_SPW = _S // _NW
_CS = 4
_C = _B * _CS
_NCHUNK = _SPW // _CS
_NBUF = 7
_NPOS = 3
_LOOK = 5

_mesh = plsc.VectorSubcoreMesh(core_axis_name="c", subcore_axis_name="s")


@functools.partial(
    pl.kernel,
    mesh=_mesh,
    out_type=jax.ShapeDtypeStruct((_B * _S, _D), jnp.float32),
    scratch_types=[
        pltpu.VMEM((_B, _SPW), jnp.int32),
        pltpu.VMEM((_NBUF * _B, _CS, _D), jnp.float32),
        pltpu.VMEM((_NPOS, _CS, _D), jnp.float32),
    ] + [pltpu.SemaphoreType.DMA] * (2 * _NBUF + _NPOS),
)
def _embed(x_hbm, tok_hbm, pos_hbm, out_hbm, idx_v, tok_v, pos_v, *sems):
    cid = lax.axis_index("c")
    sid = lax.axis_index("s")
    wid = sid * _NC + cid
    s0 = wid * _SPW

    gsems = list(sems[:_NBUF])
    wsems = list(sems[_NBUF:2 * _NBUF])
    psems = list(sems[2 * _NBUF:])

    idx_cps = [
        pltpu.async_copy(x_hbm.at[bb, pl.ds(s0, _SPW)], idx_v.at[bb],
                         psems[0])
        for bb in range(_B)
    ]
    for cp in idx_cps:
        cp.wait()

    gathers = [None] * _NCHUNK
    writes = [None] * _NCHUNK
    poss = [None] * _NCHUNK

    def start_gather(i):
        b = i % _NBUF
        gathers[i] = [
            pltpu.async_copy(
                tok_hbm.at[idx_v.at[bb, pl.ds(i * _CS, _CS)]],
                tok_v.at[b * _B + bb], gsems[b])
            for bb in range(_B)
        ]

    def start_pos(i):
        p = i % _NPOS
        poss[i] = pltpu.async_copy(
            pos_hbm.at[pl.ds(s0 + i * _CS, _CS)], pos_v.at[p], psems[p])

    for i in range(_LOOK):
        start_gather(i)
    for i in range(_NPOS):
        start_pos(i)

    for i in range(_NCHUNK):
        b = i % _NBUF
        p = i % _NPOS
        for cp in gathers[i]:
            cp.wait()
        poss[i].wait()

        def vadd_body(j, _):
            for u in range(2):
                for t in range(_CS):
                    off = j * 32 + u * 16
                    vec = pos_v[p, t, pl.ds(off, 16)]
                    for bb in range(_B):
                        plsc.addupdate(
                            tok_v.at[b * _B + bb, t, pl.ds(off, 16)], vec)
            return 0

        lax.fori_loop(0, _D // 32, vadd_body, 0)

        if i + _NPOS < _NCHUNK:
            start_pos(i + _NPOS)

        writes[i] = [
            pltpu.async_copy(
                tok_v.at[b * _B + bb],
                out_hbm.at[pl.ds(bb * _S + s0 + i * _CS, _CS)], wsems[b])
            for bb in range(_B)
        ]
        if i + _LOOK < _NCHUNK:
            if i >= _NBUF - _LOOK:
                for cp in writes[i - (_NBUF - _LOOK)]:
                    cp.wait()
            start_gather(i + _LOOK)

    waited = set(
        i - (_NBUF - _LOOK)
        for i in range(_NCHUNK)
        if i + _LOOK < _NCHUNK and i >= _NBUF - _LOOK
    )
    for i in range(_NCHUNK):
        if i not in waited:
            for cp in writes[i]:
                cp.wait()


@jax.jit
def kernel(x, token_table, pos_table):
    out = _embed(x.astype(jnp.int32), token_table, pos_table)
    return out.reshape(_B, _S, _D)


# speedup vs baseline: 1.0905x; 1.0905x over previous
import functools

import jax
import jax.numpy as jnp
from jax import lax
from jax.experimental import pallas as pl
from jax.experimental.pallas import tpu as pltpu
from jax.experimental.pallas import tpu_sc as plsc

_B, _S, _D = 4, 2048, 1024
_NC, _NS = 2, 16
_NW = _NC * _NS
_SPW = _S // _NW
_CS = 4
_C = _B * _CS
_NCHUNK = _SPW // _CS
_NBUF = 7
_NPOS = 3
_LOOK = 5

_mesh = plsc.VectorSubcoreMesh(core_axis_name="c", subcore_axis_name="s")


@functools.partial(
    pl.kernel,
    mesh=_mesh,
    out_type=jax.ShapeDtypeStruct((_B * _S, _D), jnp.float32),
    scratch_types=[
        pltpu.VMEM((_B, _SPW), jnp.int32),
        pltpu.VMEM((_NBUF * _B, _CS, _D), jnp.float32),
        pltpu.VMEM((_NPOS, _CS, _D), jnp.float32),
    ] + [pltpu.SemaphoreType.DMA] * (2 * _NBUF + _NPOS),
)
def _embed(x_hbm, tok_hbm, pos_hbm, out_hbm, idx_v, tok_v, pos_v, *sems):
    cid = lax.axis_index("c")
    sid = lax.axis_index("s")
    wid = sid * _NC + cid
    s0 = wid * _SPW

    gsems = list(sems[:_NBUF])
    wsems = list(sems[_NBUF:2 * _NBUF])
    psems = list(sems[2 * _NBUF:])

    idx_cps = [
        pltpu.async_copy(x_hbm.at[bb, pl.ds(s0, _SPW)], idx_v.at[bb],
                         psems[0])
        for bb in range(_B)
    ]
    for cp in idx_cps:
        cp.wait()

    gathers = [None] * _NCHUNK
    writes = [None] * _NCHUNK
    poss = [None] * _NCHUNK

    def start_gather(i):
        b = i % _NBUF
        gathers[i] = [
            pltpu.async_copy(
                tok_hbm.at[idx_v.at[bb, pl.ds(i * _CS, _CS)]],
                tok_v.at[b * _B + bb], gsems[b])
            for bb in range(_B)
        ]

    def start_pos(i):
        p = i % _NPOS
        poss[i] = pltpu.async_copy(
            pos_hbm.at[pl.ds(s0 + i * _CS, _CS)], pos_v.at[p], psems[p])

    for i in range(_LOOK):
        start_gather(i)
    for i in range(_NPOS):
        start_pos(i)

    for i in range(_NCHUNK):
        b = i % _NBUF
        p = i % _NPOS
        for cp in gathers[i]:
            cp.wait()
        poss[i].wait()

        def vadd_body(j, _):
            for t in range(_CS):
                vec = pos_v[p, t, pl.ds(j * 16, 16)]
                for bb in range(_B):
                    plsc.addupdate(
                        tok_v.at[b * _B + bb, t, pl.ds(j * 16, 16)], vec)
            return 0

        lax.fori_loop(0, _D // 16, vadd_body, 0)

        if i + _NPOS < _NCHUNK:
            start_pos(i + _NPOS)

        writes[i] = [
            pltpu.async_copy(
                tok_v.at[b * _B + bb],
                out_hbm.at[pl.ds(bb * _S + s0 + i * _CS, _CS)], wsems[b])
            for bb in range(_B)
        ]
        if i + _LOOK < _NCHUNK:
            if i >= _NBUF - _LOOK:
                for cp in writes[i - (_NBUF - _LOOK)]:
                    cp.wait()
            start_gather(i + _LOOK)

    waited = set(
        i - (_NBUF - _LOOK)
        for i in range(_NCHUNK)
        if i + _LOOK < _NCHUNK and i >= _NBUF - _LOOK
    )
    for i in range(_NCHUNK):
        if i not in waited:
            for cp in writes[i]:
                cp.wait()


@jax.jit
def kernel(x, token_table, pos_table):
    out = _embed(x.astype(jnp.int32), token_table, pos_table)
    return out.reshape(_B, _S, _D)
